# baseline (speedup 1.0000x reference)
import functools

import jax
import jax.numpy as jnp
from jax import lax
from jax.experimental import pallas as pl
from jax.experimental.pallas import tpu as pltpu
from jax.experimental.pallas import tpu_sc as plsc

F32 = jnp.float32

NN = 4096
NE = 65536
NG = 64
NPGR = 64
DI = 128
DE = 16
DH = 128

EROWS = NE // 128
NWORK = 32
ROWS_PER_W = EROWS // NWORK


def _dot(a, b):
    return jnp.dot(a, b, preferred_element_type=F32)


def _h0_body(x_ref, w_ref, b_ref, o_ref):
    o_ref[...] = _dot(x_ref[...], w_ref[...]) + b_ref[...]


def _tc_h0(x, w, b):
    return pl.pallas_call(
        _h0_body,
        out_shape=jax.ShapeDtypeStruct((NN, DH), F32),
    )(x, w, b.reshape(1, DH))


def _bond_body(ea_ref, bw_ref, bb_ref, o_ref):
    o_ref[...] = _dot(ea_ref[...], bw_ref[...]) + bb_ref[...]


def _tc_bond(ea, bw, bb):
    nblk = 16
    rows = NE // nblk
    full = lambda s: pl.BlockSpec(s, lambda i: (0, 0))
    return pl.pallas_call(
        _bond_body,
        grid=(nblk,),
        in_specs=[
            pl.BlockSpec((rows, DE), lambda i: (i, 0)),
            full((DE, DH)), full((1, DH)),
        ],
        out_specs=pl.BlockSpec((rows, DH), lambda i: (i, 0)),
        out_shape=jax.ShapeDtypeStruct((NE, DH), F32),
    )(ea, bw, bb.reshape(1, DH))


def _edge_body(e_ref, w1_ref, b1_ref, w2_ref, b2_ref, o_ref):
    t = jnp.maximum(_dot(e_ref[...], w1_ref[...]) + b1_ref[...], 0.0)
    o_ref[...] = _dot(t, w2_ref[...]) + b2_ref[...]


def _tc_edge_mlp(e, w1, b1, w2, b2):
    nblk = 32
    rows = NE // nblk
    full = lambda s: pl.BlockSpec(s, lambda i: (0, 0))
    return pl.pallas_call(
        _edge_body,
        grid=(nblk,),
        in_specs=[
            pl.BlockSpec((rows, DH), lambda i: (i, 0)),
            full((DH, DH)), full((1, DH)), full((DH, DH)), full((1, DH)),
        ],
        out_specs=pl.BlockSpec((rows, DH), lambda i: (i, 0)),
        out_shape=jax.ShapeDtypeStruct((NE, DH), F32),
    )(e, w1, b1.reshape(1, DH), w2, b2.reshape(1, DH))


def _post_body(h_ref, a_ref, eps_ref, w1_ref, b1_ref, w2_ref, b2_ref,
               g_ref, be_ref, o_ref):
    aggr = a_ref[0] + a_ref[1]
    z = (1.0 + eps_ref[0, 0]) * h_ref[...] + aggr
    z = jnp.maximum(_dot(z, w1_ref[...]) + b1_ref[...], 0.0)
    z = _dot(z, w2_ref[...]) + b2_ref[...]
    mu = jnp.mean(z, axis=0, keepdims=True)
    var = jnp.mean((z - mu) * (z - mu), axis=0, keepdims=True)
    zn = (z - mu) * lax.rsqrt(var + 1e-5) * g_ref[...] + be_ref[...]
    o_ref[...] = jnp.maximum(zn, 0.0)


def _tc_post(h, aggr2, eps, w1, b1, w2, b2, gamma, beta):
    return pl.pallas_call(
        _post_body,
        out_shape=jax.ShapeDtypeStruct((NN, DH), F32),
    )(h, aggr2, eps.reshape(1, 1), w1, b1.reshape(1, DH), w2,
      b2.reshape(1, DH), gamma.reshape(1, DH), beta.reshape(1, DH))


def _pairs_body(h_ref, f_ref, w1a_ref, w1b_ref, w256_ref, b1_ref,
                w2t_ref, b2_ref, o_ref):
    hg = h_ref[...]
    a = _dot(hg, w1a_ref[...]) + b1_ref[...]
    b = _dot(hg, w1b_ref[...])
    f = jnp.minimum(f_ref[0, :, :NPGR] + f_ref[1, :, :NPGR], 1.0)
    q = (a[:, None, :] + b[None, :, :]
         + f[:, :, None] * w256_ref[...][None])
    q = jnp.maximum(q, 0.0)
    o = jnp.sum(q * w2t_ref[...][None], axis=-1) + b2_ref[0, 0]
    o_ref[...] = o[None]


def _tc_pairs(h, flag2, w1, b1, w2, b2):
    w1a = w1[:DH]
    w1b = w1[DH:2 * DH]
    w256 = w1[2 * DH:2 * DH + 1]
    full = lambda s: pl.BlockSpec(s, lambda g: (0, 0))
    return pl.pallas_call(
        _pairs_body,
        grid=(NG,),
        in_specs=[
            pl.BlockSpec((NPGR, DH), lambda g: (g, 0)),
            pl.BlockSpec((2, NPGR, 128), lambda g: (0, g, 0)),
            full((DH, DH)), full((DH, DH)), full((1, DH)), full((1, DH)),
            full((1, DH)), full((1, 1)),
        ],
        out_specs=pl.BlockSpec((1, NPGR, NPGR), lambda g: (g, 0, 0)),
        out_shape=jax.ShapeDtypeStruct((NG, NPGR, NPGR), F32),
    )(h, flag2, w1a, w1b, w256, b1.reshape(1, DH), w2.reshape(1, DH),
      b2.reshape(1, 1))


def _sc_msg_body(h_hbm, ee_hbm, src_hbm, dst_hbm, out_hbm,
                 sidx, didx, hrows0, erows0, zbuf, acc, gsem0, esem0):
    cid = lax.axis_index("c")
    sid = lax.axis_index("s")
    wedge = sid * 2 + cid

    zv = jnp.zeros((16,), F32)

    def _zb(i, _):
        for k in range(8):
            zbuf[i, pl.ds(k * 16, 16)] = zv
        return 0

    lax.fori_loop(0, 16, _zb, 0)
    for rep in range(16):
        pltpu.sync_copy(zbuf, acc.at[pl.ds(sid * 256 + rep * 16, 16)])
    plsc.subcore_barrier()

    pltpu.sync_copy(src_hbm.at[pl.ds(wedge * ROWS_PER_W, ROWS_PER_W)], sidx)
    pltpu.sync_copy(dst_hbm.at[pl.ds(wedge * ROWS_PER_W, ROWS_PER_W)], didx)

    for c16 in range(ROWS_PER_W):
        base = wedge * (ROWS_PER_W * 128) + c16 * 128
        cg = pltpu.async_copy(h_hbm.at[sidx.at[c16]], hrows0, gsem0)
        ce = pltpu.async_copy(ee_hbm.at[pl.ds(base, 128)], erows0, esem0)
        cg.wait()
        ce.wait()

        def _mrow(r, _):
            for k in range(8):
                sl = pl.ds(k * 16, 16)
                erows0[r, sl] = jnp.maximum(hrows0[r, sl] + erows0[r, sl], 0.0)
            return 0

        lax.fori_loop(0, 128, _mrow, 0, unroll=4)
        pltpu.sync_copy(erows0, acc.at[didx.at[c16]], add=True)

    plsc.subcore_barrier()
    for rep in range(2):
        r0 = sid * 256 + rep * 128
        pltpu.sync_copy(acc.at[pl.ds(r0, 128)],
                        out_hbm.at[cid].at[pl.ds(r0, 128)])


def _sc_flag_body(src_hbm, dst_hbm, out_hbm, sidx, didx, obuf, zbuf, acc):
    cid = lax.axis_index("c")
    sid = lax.axis_index("s")
    wedge = sid * 2 + cid

    zv = jnp.zeros((16,), F32)

    def _zo(i, _):
        for k in range(8):
            obuf[i, pl.ds(k * 16, 16)] = zv
        return 0

    lax.fori_loop(0, 128, _zo, 0)

    def _zz(i, _):
        for k in range(8):
            zbuf[i, pl.ds(k * 16, 16)] = zv
        return 0

    lax.fori_loop(0, 16, _zz, 0)
    for rep in range(16):
        pltpu.sync_copy(zbuf, acc.at[pl.ds(sid * 256 + rep * 16, 16)])
    plsc.subcore_barrier()

    pltpu.sync_copy(src_hbm.at[pl.ds(wedge * ROWS_PER_W, ROWS_PER_W)], sidx)
    pltpu.sync_copy(dst_hbm.at[pl.ds(wedge * ROWS_PER_W, ROWS_PER_W)], didx)

    ones = jnp.full((16,), 1.0, F32)
    m63 = jnp.full((16,), 63, jnp.int32)
    lane = lax.iota(jnp.int32, 16)

    def _onehot(c, val):
        for k in range(8):
            sl = pl.ds(k * 16, 16)
            s = sidx[c, sl]
            d = didx[c, sl]
            match = (lax.shift_right_logical(s, 6)
                     == lax.shift_right_logical(d, 6))
            row = jnp.full((16,), k * 16, jnp.int32) + lane
            plsc.store_scatter(obuf, [row, d & m63], val, mask=match)

    for c16 in range(ROWS_PER_W):
        if c16 > 0:
            _onehot(c16 - 1, zv)
        _onehot(c16, ones)
        pltpu.sync_copy(obuf, acc.at[sidx.at[c16]], add=True)

    plsc.subcore_barrier()
    for rep in range(2):
        r0 = sid * 256 + rep * 128
        pltpu.sync_copy(acc.at[pl.ds(r0, 128)],
                        out_hbm.at[cid].at[pl.ds(r0, 128)])


@functools.lru_cache(maxsize=1)
def _get_sc_kernels():
    mesh = plsc.VectorSubcoreMesh(core_axis_name="c", subcore_axis_name="s")

    cparams = pltpu.CompilerParams(needs_layout_passes=False)

    sc_msg = functools.partial(
        pl.kernel,
        mesh=mesh,
        compiler_params=cparams,
        out_type=jax.ShapeDtypeStruct((2, NN, DH), F32),
        scratch_types=[
            pltpu.VMEM((ROWS_PER_W, 128), jnp.int32),
            pltpu.VMEM((ROWS_PER_W, 128), jnp.int32),
            pltpu.VMEM((128, DH), F32),
            pltpu.VMEM((128, DH), F32),
            pltpu.VMEM((16, DH), F32),
            pltpu.VMEM_SHARED((NN, DH), F32),
            pltpu.SemaphoreType.DMA,
            pltpu.SemaphoreType.DMA,
        ],
    )(_sc_msg_body)

    sc_flag = functools.partial(
        pl.kernel,
        mesh=mesh,
        compiler_params=cparams,
        out_type=jax.ShapeDtypeStruct((2, NN, 128), F32),
        scratch_types=[
            pltpu.VMEM((ROWS_PER_W, 128), jnp.int32),
            pltpu.VMEM((ROWS_PER_W, 128), jnp.int32),
            pltpu.VMEM((128, 128), F32),
            pltpu.VMEM((16, 128), F32),
            pltpu.VMEM_SHARED((NN, 128), F32),
        ],
    )(_sc_flag_body)

    return sc_msg, sc_flag


def kernel(x, edge_index, edge_attr, params):
    src2d = edge_index[0].reshape(EROWS, 128)
    dst2d = edge_index[1].reshape(EROWS, 128)

    sc_msg, sc_flag = _get_sc_kernels()
    flag2 = sc_flag(src2d, dst2d)

    h = _tc_h0(x, params['atom_W'], params['atom_b'])
    e = _tc_bond(edge_attr, params['bond_W'], params['bond_b'])
    ees = [_tc_edge_mlp(e, lp['be_W1'], lp['be_b1'], lp['be_W2'], lp['be_b2'])
           for lp in params['layers']]
    for lp, ee in zip(params['layers'], ees):
        aggr2 = sc_msg(h, ee, src2d, dst2d)
        h = _tc_post(h, aggr2, lp['eps'], lp['nn_W1'], lp['nn_b1'],
                     lp['nn_W2'], lp['nn_b2'], lp['bn_gamma'], lp['bn_beta'])

    out3 = _tc_pairs(h, flag2, params['mlp_W1'], params['mlp_b1'],
                     params['mlp_W2'], params['mlp_b2'])
    return out3.reshape(NG * NPGR * NPGR, 1)


# speedup vs baseline: 1.3802x; 1.3802x over previous
import functools

import jax
import jax.numpy as jnp
from jax import lax
from jax.experimental import pallas as pl
from jax.experimental.pallas import tpu as pltpu
from jax.experimental.pallas import tpu_sc as plsc

F32 = jnp.float32

NN = 4096
NE = 65536
NG = 64
NPGR = 64
DI = 128
DE = 16
DH = 128

EROWS = NE // 128
NWORK = 32
ROWS_PER_W = EROWS // NWORK


def _dot(a, b):
    return jnp.dot(a, b, preferred_element_type=F32)


def _h0_body(x_ref, w_ref, b_ref, o_ref):
    o_ref[...] = _dot(x_ref[...], w_ref[...]) + b_ref[...]


def _tc_h0(x, w, b):
    return pl.pallas_call(
        _h0_body,
        out_shape=jax.ShapeDtypeStruct((NN, DH), F32),
    )(x, w, b.reshape(1, DH))


def _bond_body(ea_ref, bw_ref, bb_ref, o_ref):
    o_ref[...] = _dot(ea_ref[...], bw_ref[...]) + bb_ref[...]


def _tc_bond(ea, bw, bb):
    nblk = 16
    rows = NE // nblk
    full = lambda s: pl.BlockSpec(s, lambda i: (0, 0))
    return pl.pallas_call(
        _bond_body,
        grid=(nblk,),
        in_specs=[
            pl.BlockSpec((rows, DE), lambda i: (i, 0)),
            full((DE, DH)), full((1, DH)),
        ],
        out_specs=pl.BlockSpec((rows, DH), lambda i: (i, 0)),
        out_shape=jax.ShapeDtypeStruct((NE, DH), F32),
    )(ea, bw, bb.reshape(1, DH))


def _edge_body(e_ref, w1_ref, b1_ref, w2_ref, b2_ref, o_ref):
    t = jnp.maximum(_dot(e_ref[...], w1_ref[...]) + b1_ref[...], 0.0)
    o_ref[...] = _dot(t, w2_ref[...]) + b2_ref[...]


def _tc_edge_mlp(e, w1, b1, w2, b2):
    nblk = 32
    rows = NE // nblk
    full = lambda s: pl.BlockSpec(s, lambda i: (0, 0))
    return pl.pallas_call(
        _edge_body,
        grid=(nblk,),
        in_specs=[
            pl.BlockSpec((rows, DH), lambda i: (i, 0)),
            full((DH, DH)), full((1, DH)), full((DH, DH)), full((1, DH)),
        ],
        out_specs=pl.BlockSpec((rows, DH), lambda i: (i, 0)),
        out_shape=jax.ShapeDtypeStruct((NE, DH), F32),
    )(e, w1, b1.reshape(1, DH), w2, b2.reshape(1, DH))


def _post_body(h_ref, a_ref, eps_ref, w1_ref, b1_ref, w2_ref, b2_ref,
               g_ref, be_ref, o_ref):
    aggr = a_ref[0] + a_ref[1]
    z = (1.0 + eps_ref[0, 0]) * h_ref[...] + aggr
    z = jnp.maximum(_dot(z, w1_ref[...]) + b1_ref[...], 0.0)
    z = _dot(z, w2_ref[...]) + b2_ref[...]
    mu = jnp.mean(z, axis=0, keepdims=True)
    var = jnp.mean((z - mu) * (z - mu), axis=0, keepdims=True)
    zn = (z - mu) * lax.rsqrt(var + 1e-5) * g_ref[...] + be_ref[...]
    o_ref[...] = jnp.maximum(zn, 0.0)


def _tc_post(h, aggr2, eps, w1, b1, w2, b2, gamma, beta):
    return pl.pallas_call(
        _post_body,
        out_shape=jax.ShapeDtypeStruct((NN, DH), F32),
    )(h, aggr2, eps.reshape(1, 1), w1, b1.reshape(1, DH), w2,
      b2.reshape(1, DH), gamma.reshape(1, DH), beta.reshape(1, DH))


def _pairs_body(h_ref, f_ref, w1a_ref, w1b_ref, w256_ref, b1_ref,
                w2t_ref, b2_ref, o_ref):
    hg = h_ref[...]
    a = _dot(hg, w1a_ref[...]) + b1_ref[...]
    b = _dot(hg, w1b_ref[...])
    f = jnp.minimum(f_ref[0, :, :NPGR] + f_ref[1, :, :NPGR], 1.0)
    q = (a[:, None, :] + b[None, :, :]
         + f[:, :, None] * w256_ref[...][None])
    q = jnp.maximum(q, 0.0)
    o = jnp.sum(q * w2t_ref[...][None], axis=-1) + b2_ref[0, 0]
    o_ref[...] = o[None]


def _tc_pairs(h, flag2, w1, b1, w2, b2):
    w1a = w1[:DH]
    w1b = w1[DH:2 * DH]
    w256 = w1[2 * DH:2 * DH + 1]
    full = lambda s: pl.BlockSpec(s, lambda g: (0, 0))
    return pl.pallas_call(
        _pairs_body,
        grid=(NG,),
        in_specs=[
            pl.BlockSpec((NPGR, DH), lambda g: (g, 0)),
            pl.BlockSpec((2, NPGR, 128), lambda g: (0, g, 0)),
            full((DH, DH)), full((DH, DH)), full((1, DH)), full((1, DH)),
            full((1, DH)), full((1, 1)),
        ],
        out_specs=pl.BlockSpec((1, NPGR, NPGR), lambda g: (g, 0, 0)),
        out_shape=jax.ShapeDtypeStruct((NG, NPGR, NPGR), F32),
    )(h, flag2, w1a, w1b, w256, b1.reshape(1, DH), w2.reshape(1, DH),
      b2.reshape(1, 1))


def _sc_msg_body(h_hbm, ee_hbm, src_hbm, dst_hbm, out_hbm,
                 sidx, didx, hrows0, erows0, zbuf, acc, gsem0, esem0):
    cid = lax.axis_index("c")
    sid = lax.axis_index("s")
    wedge = sid * 2 + cid

    zv = jnp.zeros((16,), F32)

    def _zb(i, _):
        for k in range(8):
            zbuf[i, pl.ds(k * 16, 16)] = zv
        return 0

    lax.fori_loop(0, 16, _zb, 0)
    for rep in range(16):
        pltpu.sync_copy(zbuf, acc.at[pl.ds(sid * 256 + rep * 16, 16)])
    plsc.subcore_barrier()

    pltpu.sync_copy(src_hbm.at[pl.ds(wedge * ROWS_PER_W, ROWS_PER_W)], sidx)
    pltpu.sync_copy(dst_hbm.at[pl.ds(wedge * ROWS_PER_W, ROWS_PER_W)], didx)

    for c16 in range(ROWS_PER_W):
        base = wedge * (ROWS_PER_W * 128) + c16 * 128
        cg = pltpu.async_copy(h_hbm.at[sidx.at[c16]], hrows0, gsem0)
        ce = pltpu.async_copy(ee_hbm.at[pl.ds(base, 128)], erows0, esem0)
        cg.wait()
        ce.wait()

        def _mrow(r, _):
            for k in range(8):
                sl = pl.ds(k * 16, 16)
                erows0[r, sl] = jnp.maximum(hrows0[r, sl] + erows0[r, sl], 0.0)
            return 0

        lax.fori_loop(0, 128, _mrow, 0)
        pltpu.sync_copy(erows0, acc.at[didx.at[c16]], add=True)

    plsc.subcore_barrier()
    for rep in range(2):
        r0 = sid * 256 + rep * 128
        pltpu.sync_copy(acc.at[pl.ds(r0, 128)],
                        out_hbm.at[cid].at[pl.ds(r0, 128)])


def _sc_flag_body(src_hbm, dst_hbm, out_hbm, sidx, didx, obuf, zbuf, acc):
    cid = lax.axis_index("c")
    sid = lax.axis_index("s")
    wedge = sid * 2 + cid

    zv = jnp.zeros((16,), F32)

    def _zo(i, _):
        for k in range(8):
            obuf[i, pl.ds(k * 16, 16)] = zv
        return 0

    lax.fori_loop(0, 128, _zo, 0)

    def _zz(i, _):
        for k in range(8):
            zbuf[i, pl.ds(k * 16, 16)] = zv
        return 0

    lax.fori_loop(0, 16, _zz, 0)
    for rep in range(16):
        pltpu.sync_copy(zbuf, acc.at[pl.ds(sid * 256 + rep * 16, 16)])
    plsc.subcore_barrier()

    pltpu.sync_copy(src_hbm.at[pl.ds(wedge * ROWS_PER_W, ROWS_PER_W)], sidx)
    pltpu.sync_copy(dst_hbm.at[pl.ds(wedge * ROWS_PER_W, ROWS_PER_W)], didx)

    ones = jnp.full((16,), 1.0, F32)
    m63 = jnp.full((16,), 63, jnp.int32)
    lane = lax.iota(jnp.int32, 16)

    def _onehot(c, val):
        for k in range(8):
            sl = pl.ds(k * 16, 16)
            s = sidx[c, sl]
            d = didx[c, sl]
            match = (lax.shift_right_logical(s, 6)
                     == lax.shift_right_logical(d, 6))
            row = jnp.full((16,), k * 16, jnp.int32) + lane
            plsc.store_scatter(obuf, [row, d & m63], val, mask=match)

    for c16 in range(ROWS_PER_W):
        if c16 > 0:
            _onehot(c16 - 1, zv)
        _onehot(c16, ones)
        pltpu.sync_copy(obuf, acc.at[sidx.at[c16]], add=True)

    plsc.subcore_barrier()
    for rep in range(2):
        r0 = sid * 256 + rep * 128
        pltpu.sync_copy(acc.at[pl.ds(r0, 128)],
                        out_hbm.at[cid].at[pl.ds(r0, 128)])


@functools.lru_cache(maxsize=1)
def _get_sc_kernels():
    mesh = plsc.VectorSubcoreMesh(core_axis_name="c", subcore_axis_name="s")

    cparams = pltpu.CompilerParams(needs_layout_passes=False)

    sc_msg = functools.partial(
        pl.kernel,
        mesh=mesh,
        compiler_params=cparams,
        out_type=jax.ShapeDtypeStruct((2, NN, DH), F32),
        scratch_types=[
            pltpu.VMEM((ROWS_PER_W, 128), jnp.int32),
            pltpu.VMEM((ROWS_PER_W, 128), jnp.int32),
            pltpu.VMEM((128, DH), F32),
            pltpu.VMEM((128, DH), F32),
            pltpu.VMEM((16, DH), F32),
            pltpu.VMEM_SHARED((NN, DH), F32),
            pltpu.SemaphoreType.DMA,
            pltpu.SemaphoreType.DMA,
        ],
    )(_sc_msg_body)

    sc_flag = functools.partial(
        pl.kernel,
        mesh=mesh,
        compiler_params=cparams,
        out_type=jax.ShapeDtypeStruct((2, NN, 128), F32),
        scratch_types=[
            pltpu.VMEM((ROWS_PER_W, 128), jnp.int32),
            pltpu.VMEM((ROWS_PER_W, 128), jnp.int32),
            pltpu.VMEM((128, 128), F32),
            pltpu.VMEM((16, 128), F32),
            pltpu.VMEM_SHARED((NN, 128), F32),
        ],
    )(_sc_flag_body)

    return sc_msg, sc_flag


def kernel(x, edge_index, edge_attr, params):
    src2d = edge_index[0].reshape(EROWS, 128)
    dst2d = edge_index[1].reshape(EROWS, 128)

    sc_msg, sc_flag = _get_sc_kernels()
    flag2 = sc_flag(src2d, dst2d)

    h = _tc_h0(x, params['atom_W'], params['atom_b'])
    e = _tc_bond(edge_attr, params['bond_W'], params['bond_b'])
    ees = [_tc_edge_mlp(e, lp['be_W1'], lp['be_b1'], lp['be_W2'], lp['be_b2'])
           for lp in params['layers']]
    for lp, ee in zip(params['layers'], ees):
        aggr2 = sc_msg(h, ee, src2d, dst2d)
        h = _tc_post(h, aggr2, lp['eps'], lp['nn_W1'], lp['nn_b1'],
                     lp['nn_W2'], lp['nn_b2'], lp['bn_gamma'], lp['bn_beta'])

    out3 = _tc_pairs(h, flag2, params['mlp_W1'], params['mlp_b1'],
                     params['mlp_W2'], params['mlp_b2'])
    return out3.reshape(NG * NPGR * NPGR, 1)


# speedup vs baseline: 1.4513x; 1.0515x over previous
import functools

import jax
import jax.numpy as jnp
from jax import lax
from jax.experimental import pallas as pl
from jax.experimental.pallas import tpu as pltpu
from jax.experimental.pallas import tpu_sc as plsc

F32 = jnp.float32

NN = 4096
NE = 65536
NG = 64
NPGR = 64
DI = 128
DE = 16
DH = 128

EROWS = NE // 128
NWORK = 32
ROWS_PER_W = EROWS // NWORK


def _dot(a, b):
    return jnp.dot(a, b, preferred_element_type=F32)


def _h0_body(x_ref, w_ref, b_ref, o_ref):
    o_ref[...] = _dot(x_ref[...], w_ref[...]) + b_ref[...]


def _tc_h0(x, w, b):
    return pl.pallas_call(
        _h0_body,
        out_shape=jax.ShapeDtypeStruct((NN, DH), F32),
    )(x, w, b.reshape(1, DH))


def _edge2_body(ea_ref, bw_ref, bb_ref, w10_ref, b10_ref, w20_ref, b20_ref,
                w11_ref, b11_ref, w21_ref, b21_ref, o0_ref, o1_ref):
    e = _dot(ea_ref[...], bw_ref[...]) + bb_ref[...]
    t0 = jnp.maximum(_dot(e, w10_ref[...]) + b10_ref[...], 0.0)
    o0_ref[...] = _dot(t0, w20_ref[...]) + b20_ref[...]
    t1 = jnp.maximum(_dot(e, w11_ref[...]) + b11_ref[...], 0.0)
    o1_ref[...] = _dot(t1, w21_ref[...]) + b21_ref[...]


def _tc_edge_mlp2(ea, bw, bb, lp0, lp1):
    nblk = 32
    rows = NE // nblk
    full = lambda s: pl.BlockSpec(s, lambda i: (0, 0))
    ospec = pl.BlockSpec((rows, DH), lambda i: (i, 0))
    oshape = jax.ShapeDtypeStruct((NE, DH), F32)
    return pl.pallas_call(
        _edge2_body,
        grid=(nblk,),
        in_specs=[
            pl.BlockSpec((rows, DE), lambda i: (i, 0)),
            full((DE, DH)), full((1, DH)),
            full((DH, DH)), full((1, DH)), full((DH, DH)), full((1, DH)),
            full((DH, DH)), full((1, DH)), full((DH, DH)), full((1, DH)),
        ],
        out_specs=[ospec, ospec],
        out_shape=[oshape, oshape],
    )(ea, bw, bb.reshape(1, DH),
      lp0['be_W1'], lp0['be_b1'].reshape(1, DH),
      lp0['be_W2'], lp0['be_b2'].reshape(1, DH),
      lp1['be_W1'], lp1['be_b1'].reshape(1, DH),
      lp1['be_W2'], lp1['be_b2'].reshape(1, DH))


def _post_body(h_ref, a_ref, eps_ref, w1_ref, b1_ref, w2_ref, b2_ref,
               g_ref, be_ref, o_ref):
    aggr = a_ref[0] + a_ref[1]
    z = (1.0 + eps_ref[0, 0]) * h_ref[...] + aggr
    z = jnp.maximum(_dot(z, w1_ref[...]) + b1_ref[...], 0.0)
    z = _dot(z, w2_ref[...]) + b2_ref[...]
    mu = jnp.mean(z, axis=0, keepdims=True)
    var = jnp.mean((z - mu) * (z - mu), axis=0, keepdims=True)
    zn = (z - mu) * lax.rsqrt(var + 1e-5) * g_ref[...] + be_ref[...]
    o_ref[...] = jnp.maximum(zn, 0.0)


def _tc_post(h, aggr2, eps, w1, b1, w2, b2, gamma, beta):
    return pl.pallas_call(
        _post_body,
        out_shape=jax.ShapeDtypeStruct((NN, DH), F32),
    )(h, aggr2, eps.reshape(1, 1), w1, b1.reshape(1, DH), w2,
      b2.reshape(1, DH), gamma.reshape(1, DH), beta.reshape(1, DH))


def _pairs_body(h_ref, f_ref, w1a_ref, w1b_ref, w256_ref, b1_ref,
                w2t_ref, b2_ref, o_ref):
    hg = h_ref[...]
    a = _dot(hg, w1a_ref[...]) + b1_ref[...]
    b = _dot(hg, w1b_ref[...])
    f = jnp.minimum(f_ref[0, :, :NPGR] + f_ref[1, :, :NPGR], 1.0)
    q = (a[:, None, :] + b[None, :, :]
         + f[:, :, None] * w256_ref[...][None])
    q = jnp.maximum(q, 0.0)
    o = jnp.sum(q * w2t_ref[...][None], axis=-1) + b2_ref[0, 0]
    o_ref[...] = o[None]


def _tc_pairs(h, flag2, w1, b1, w2, b2):
    w1a = w1[:DH]
    w1b = w1[DH:2 * DH]
    w256 = w1[2 * DH:2 * DH + 1]
    full = lambda s: pl.BlockSpec(s, lambda g: (0, 0))
    return pl.pallas_call(
        _pairs_body,
        grid=(NG,),
        in_specs=[
            pl.BlockSpec((NPGR, DH), lambda g: (g, 0)),
            pl.BlockSpec((2, NPGR, 128), lambda g: (0, g, 0)),
            full((DH, DH)), full((DH, DH)), full((1, DH)), full((1, DH)),
            full((1, DH)), full((1, 1)),
        ],
        out_specs=pl.BlockSpec((1, NPGR, NPGR), lambda g: (g, 0, 0)),
        out_shape=jax.ShapeDtypeStruct((NG, NPGR, NPGR), F32),
    )(h, flag2, w1a, w1b, w256, b1.reshape(1, DH), w2.reshape(1, DH),
      b2.reshape(1, 1))


def _sc_msg_body(h_hbm, ee_hbm, src_hbm, dst_hbm, out_hbm,
                 sidx, didx, hrows0, erows0, zbuf, acc, gsem0, esem0):
    cid = lax.axis_index("c")
    sid = lax.axis_index("s")
    wedge = sid * 2 + cid

    zv = jnp.zeros((16,), F32)

    def _zb(i, _):
        for k in range(8):
            zbuf[i, pl.ds(k * 16, 16)] = zv
        return 0

    lax.fori_loop(0, 16, _zb, 0)
    for rep in range(16):
        pltpu.sync_copy(zbuf, acc.at[pl.ds(sid * 256 + rep * 16, 16)])
    plsc.subcore_barrier()

    pltpu.sync_copy(src_hbm.at[pl.ds(wedge * ROWS_PER_W, ROWS_PER_W)], sidx)
    pltpu.sync_copy(dst_hbm.at[pl.ds(wedge * ROWS_PER_W, ROWS_PER_W)], didx)

    for c16 in range(ROWS_PER_W):
        base = wedge * (ROWS_PER_W * 128) + c16 * 128
        cg = pltpu.async_copy(h_hbm.at[sidx.at[c16]], hrows0, gsem0)
        ce = pltpu.async_copy(ee_hbm.at[pl.ds(base, 128)], erows0, esem0)
        cg.wait()
        ce.wait()

        def _mrow(r, _):
            for k in range(8):
                sl = pl.ds(k * 16, 16)
                erows0[r, sl] = jnp.maximum(hrows0[r, sl] + erows0[r, sl], 0.0)
            return 0

        lax.fori_loop(0, 128, _mrow, 0)
        pltpu.sync_copy(erows0, acc.at[didx.at[c16]], add=True)

    plsc.subcore_barrier()
    for rep in range(2):
        r0 = sid * 256 + rep * 128
        pltpu.sync_copy(acc.at[pl.ds(r0, 128)],
                        out_hbm.at[cid].at[pl.ds(r0, 128)])


def _sc_flag_body(src_hbm, dst_hbm, out_hbm, sidx, didx, obuf, zbuf, acc):
    cid = lax.axis_index("c")
    sid = lax.axis_index("s")
    wedge = sid * 2 + cid

    zv = jnp.zeros((16,), F32)

    def _zo(i, _):
        for k in range(8):
            obuf[i, pl.ds(k * 16, 16)] = zv
        return 0

    lax.fori_loop(0, 128, _zo, 0)

    def _zz(i, _):
        for k in range(8):
            zbuf[i, pl.ds(k * 16, 16)] = zv
        return 0

    lax.fori_loop(0, 16, _zz, 0)
    for rep in range(16):
        pltpu.sync_copy(zbuf, acc.at[pl.ds(sid * 256 + rep * 16, 16)])
    plsc.subcore_barrier()

    pltpu.sync_copy(src_hbm.at[pl.ds(wedge * ROWS_PER_W, ROWS_PER_W)], sidx)
    pltpu.sync_copy(dst_hbm.at[pl.ds(wedge * ROWS_PER_W, ROWS_PER_W)], didx)

    ones = jnp.full((16,), 1.0, F32)
    m63 = jnp.full((16,), 63, jnp.int32)
    lane = lax.iota(jnp.int32, 16)

    def _onehot(c, val):
        for k in range(8):
            sl = pl.ds(k * 16, 16)
            s = sidx[c, sl]
            d = didx[c, sl]
            match = (lax.shift_right_logical(s, 6)
                     == lax.shift_right_logical(d, 6))
            row = jnp.full((16,), k * 16, jnp.int32) + lane
            plsc.store_scatter(obuf, [row, d & m63], val, mask=match)

    for c16 in range(ROWS_PER_W):
        if c16 > 0:
            _onehot(c16 - 1, zv)
        _onehot(c16, ones)
        pltpu.sync_copy(obuf, acc.at[sidx.at[c16]], add=True)

    plsc.subcore_barrier()
    for rep in range(2):
        r0 = sid * 256 + rep * 128
        pltpu.sync_copy(acc.at[pl.ds(r0, 128)],
                        out_hbm.at[cid].at[pl.ds(r0, 128)])


@functools.lru_cache(maxsize=1)
def _get_sc_kernels():
    mesh = plsc.VectorSubcoreMesh(core_axis_name="c", subcore_axis_name="s")

    cparams = pltpu.CompilerParams(needs_layout_passes=False)

    sc_msg = functools.partial(
        pl.kernel,
        mesh=mesh,
        compiler_params=cparams,
        out_type=jax.ShapeDtypeStruct((2, NN, DH), F32),
        scratch_types=[
            pltpu.VMEM((ROWS_PER_W, 128), jnp.int32),
            pltpu.VMEM((ROWS_PER_W, 128), jnp.int32),
            pltpu.VMEM((128, DH), F32),
            pltpu.VMEM((128, DH), F32),
            pltpu.VMEM((16, DH), F32),
            pltpu.VMEM_SHARED((NN, DH), F32),
            pltpu.SemaphoreType.DMA,
            pltpu.SemaphoreType.DMA,
        ],
    )(_sc_msg_body)

    sc_flag = functools.partial(
        pl.kernel,
        mesh=mesh,
        compiler_params=cparams,
        out_type=jax.ShapeDtypeStruct((2, NN, 128), F32),
        scratch_types=[
            pltpu.VMEM((ROWS_PER_W, 128), jnp.int32),
            pltpu.VMEM((ROWS_PER_W, 128), jnp.int32),
            pltpu.VMEM((128, 128), F32),
            pltpu.VMEM((16, 128), F32),
            pltpu.VMEM_SHARED((NN, 128), F32),
        ],
    )(_sc_flag_body)

    return sc_msg, sc_flag


def kernel(x, edge_index, edge_attr, params):
    src2d = edge_index[0].reshape(EROWS, 128)
    dst2d = edge_index[1].reshape(EROWS, 128)

    sc_msg, sc_flag = _get_sc_kernels()
    flag2 = sc_flag(src2d, dst2d)

    h = _tc_h0(x, params['atom_W'], params['atom_b'])
    ees = _tc_edge_mlp2(edge_attr, params['bond_W'], params['bond_b'],
                        params['layers'][0], params['layers'][1])
    for lp, ee in zip(params['layers'], ees):
        aggr2 = sc_msg(h, ee, src2d, dst2d)
        h = _tc_post(h, aggr2, lp['eps'], lp['nn_W1'], lp['nn_b1'],
                     lp['nn_W2'], lp['nn_b2'], lp['bn_gamma'], lp['bn_beta'])

    out3 = _tc_pairs(h, flag2, params['mlp_W1'], params['mlp_b1'],
                     params['mlp_W2'], params['mlp_b2'])
    return out3.reshape(NG * NPGR * NPGR, 1)
